# baseline (device time: 45668 ns/iter reference)
import jax
import jax.numpy as jnp
from jax import lax
from jax.experimental import pallas as pl
from jax.experimental.pallas import tpu as pltpu

N_DEV = 4


def kernel(A, B):
    m, _ = A.shape
    _, n = B.shape

    def body(a_ref, b_ref, out_ref, comm_ref, send_sems, recv_sems):
        my_pos = lax.axis_index("i")
        left = (my_pos - 1) % N_DEV
        right = (my_pos + 1) % N_DEV

        barrier_sem = pltpu.get_barrier_semaphore()
        for nbr in (left, right):
            pl.semaphore_signal(
                barrier_sem, inc=1,
                device_id=(nbr,), device_id_type=pl.DeviceIdType.MESH,
            )
        pl.semaphore_wait(barrier_sem, 2)

        partial = jnp.dot(
            a_ref[...], b_ref[...], preferred_element_type=jnp.float32
        )
        out_ref[...] = partial
        comm_ref[0] = partial

        for h in range(N_DEV - 1):
            send_slot = h % 2
            recv_slot = (h + 1) % 2
            rdma = pltpu.make_async_remote_copy(
                src_ref=comm_ref.at[send_slot],
                dst_ref=comm_ref.at[recv_slot],
                send_sem=send_sems.at[send_slot],
                recv_sem=recv_sems.at[recv_slot],
                device_id=(right,),
                device_id_type=pl.DeviceIdType.MESH,
            )
            rdma.start()
            rdma.wait()
            out_ref[...] += comm_ref[recv_slot]

    return pl.pallas_call(
        body,
        out_shape=jax.ShapeDtypeStruct((m, n), jnp.float32),
        in_specs=[
            pl.BlockSpec(memory_space=pltpu.VMEM),
            pl.BlockSpec(memory_space=pltpu.VMEM),
        ],
        out_specs=pl.BlockSpec(memory_space=pltpu.VMEM),
        scratch_shapes=[
            pltpu.VMEM((2, m, n), jnp.float32),
            pltpu.SemaphoreType.DMA((2,)),
            pltpu.SemaphoreType.DMA((2,)),
        ],
        compiler_params=pltpu.CompilerParams(collective_id=0),
    )(A, B)


# device time: 21105 ns/iter; 2.1638x vs baseline; 2.1638x over previous
import jax
import jax.numpy as jnp
from jax import lax
from jax.experimental import pallas as pl
from jax.experimental.pallas import tpu as pltpu

N_DEV = 4


def kernel(A, B):
    m, _ = A.shape
    _, n = B.shape
    nh = n // 2

    def body(a_ref, b_ref, out_ref, send_ref, recv_ref, send_sems, recv_sems):
        my_pos = lax.axis_index("i")
        pa = my_pos ^ 1
        pb = 3 - my_pos

        barrier_sem = pltpu.get_barrier_semaphore()
        for nbr in (pa, pb):
            pl.semaphore_signal(
                barrier_sem, inc=1,
                device_id=(nbr,), device_id_type=pl.DeviceIdType.MESH,
            )
        pl.semaphore_wait(barrier_sem, 2)

        send_ref[0] = jnp.dot(
            a_ref[...], b_ref[:, 0:nh], preferred_element_type=jnp.float32
        )
        send_ref[1] = jnp.dot(
            a_ref[...], b_ref[:, nh:n], preferred_element_type=jnp.float32
        )

        def exchange(slot, sem, target):
            return pltpu.make_async_remote_copy(
                src_ref=send_ref.at[slot],
                dst_ref=recv_ref.at[sem],
                send_sem=send_sems.at[sem],
                recv_sem=recv_sems.at[sem],
                device_id=(target,),
                device_id_type=pl.DeviceIdType.MESH,
            )

        r0 = exchange(0, 0, pa)
        r1 = exchange(1, 1, pb)
        r0.start()
        r1.start()
        r0.wait()
        r1.wait()
        send_ref[0] += recv_ref[0]
        send_ref[1] += recv_ref[1]

        r2 = exchange(0, 2, pb)
        r3 = exchange(1, 3, pa)
        r2.start()
        r3.start()
        r2.wait()
        r3.wait()
        out_ref[:, 0:nh] = send_ref[0] + recv_ref[2]
        out_ref[:, nh:n] = send_ref[1] + recv_ref[3]

    return pl.pallas_call(
        body,
        out_shape=jax.ShapeDtypeStruct((m, n), jnp.float32),
        in_specs=[
            pl.BlockSpec(memory_space=pltpu.VMEM),
            pl.BlockSpec(memory_space=pltpu.VMEM),
        ],
        out_specs=pl.BlockSpec(memory_space=pltpu.VMEM),
        scratch_shapes=[
            pltpu.VMEM((2, m, nh), jnp.float32),
            pltpu.VMEM((4, m, nh), jnp.float32),
            pltpu.SemaphoreType.DMA((4,)),
            pltpu.SemaphoreType.DMA((4,)),
        ],
        compiler_params=pltpu.CompilerParams(collective_id=0),
    )(A, B)
